# baseline (speedup 1.0000x reference)
import functools

import jax
import jax.numpy as jnp
from jax import lax
from jax.experimental import pallas as pl
from jax.experimental.pallas import tpu as pltpu
from jax.experimental.pallas import tpu_sc as plsc

LANES = 16
NUM_CORES = 2
NUM_SUBCORES = 16
NUM_WORKERS = NUM_CORES * NUM_SUBCORES
CPAD = 17
ISLICE = 128


def _rsqrt(x):
    i = plsc.bitcast(x, jnp.int32)
    i = 0x5F3759DF - (i >> 1)
    y = plsc.bitcast(i, jnp.float32)
    for _ in range(3):
        y = y * (1.5 - 0.5 * x * y * y)
    return y


def _make_sc_kernel(batch, bpw):
    nchunks = bpw // LANES
    nslices = bpw // ISLICE
    mesh = plsc.VectorSubcoreMesh(core_axis_name="c", subcore_axis_name="s")

    @functools.partial(
        pl.kernel,
        out_type=jax.ShapeDtypeStruct((batch, 16), jnp.float32),
        mesh=mesh,
        scratch_types=[
            pltpu.VMEM((nslices, ISLICE), jnp.int32),
            pltpu.VMEM((bpw, 16), jnp.float32),
            pltpu.VMEM((bpw, CPAD), jnp.float32),
            pltpu.VMEM((bpw, CPAD), jnp.float32),
            pltpu.SemaphoreType.DMA,
            pltpu.SemaphoreType.DMA,
        ],
        compiler_params=pltpu.CompilerParams(
            needs_layout_passes=False, use_tc_tiling_on_sc=False),
    )
    def sc_kernel(c2w_hbm, ids_hbm, tab_hbm, out_hbm,
                  idx_v, delta_v, c2w_v, out_v, sem_g, sem_c):
        wid = lax.axis_index("s") * NUM_CORES + lax.axis_index("c")
        base = wid * bpw
        lane = lax.iota(jnp.int32, LANES)

        pltpu.sync_copy(ids_hbm.at[pl.ds(wid * nslices, nslices)], idx_v)
        gathers = [
            pltpu.async_copy(tab_hbm.at[idx_v.at[j]],
                             delta_v.at[pl.ds(j * ISLICE, ISLICE)], sem_g)
            for j in range(nslices)
        ]
        load = pltpu.async_copy(
            c2w_hbm.at[pl.ds(base, bpw)], c2w_v.at[:, pl.ds(0, 16)], sem_c)
        for g in gathers:
            g.wait()
        load.wait()

        def chunk(c, carry):
            e = c * LANES + lane

            def col(ref, k):
                return plsc.load_gather(
                    ref, [e, jnp.full((LANES,), k, jnp.int32)])

            d = [col(delta_v, k) for k in range(9)]
            cw = [col(c2w_v, k) for k in range(16)]

            a10, a11, a12 = d[3] + 1.0, d[4], d[5]
            a20, a21, a22 = d[6], d[7] + 1.0, d[8]
            n1 = a10 * a10 + a11 * a11 + a12 * a12
            inv1 = _rsqrt(jnp.maximum(n1, 1e-24))
            b10, b11, b12 = a10 * inv1, a11 * inv1, a12 * inv1
            proj = b10 * a20 + b11 * a21 + b12 * a22
            u0 = a20 - proj * b10
            u1 = a21 - proj * b11
            u2 = a22 - proj * b12
            n2 = u0 * u0 + u1 * u1 + u2 * u2
            inv2 = _rsqrt(jnp.maximum(n2, 1e-24))
            b20, b21, b22 = u0 * inv2, u1 * inv2, u2 * inv2
            b30 = b11 * b22 - b12 * b21
            b31 = b12 * b20 - b10 * b22
            b32 = b10 * b21 - b11 * b20

            t = [[b10, b11, b12, d[0]],
                 [b20, b21, b22, d[1]],
                 [b30, b31, b32, d[2]]]

            for i4 in range(4):
                c0, c1, c2 = cw[4 * i4], cw[4 * i4 + 1], cw[4 * i4 + 2]
                c3 = cw[4 * i4 + 3]
                for j in range(4):
                    v = c0 * t[0][j] + c1 * t[1][j] + c2 * t[2][j]
                    if j == 3:
                        v = v + c3
                    plsc.store_scatter(
                        out_v, [e, jnp.full((LANES,), 4 * i4 + j, jnp.int32)],
                        v)
            return carry

        lax.fori_loop(0, nchunks, chunk, 0)
        pltpu.sync_copy(out_v.at[:, pl.ds(0, 16)],
                        out_hbm.at[pl.ds(base, bpw)])

    return sc_kernel


def kernel(camtoworlds, camera_ids, embeds_weight):
    batch = camtoworlds.shape[0]
    bpw = batch // NUM_WORKERS
    c2w = camtoworlds.reshape(batch, 16)
    ids2 = camera_ids.reshape(batch // ISLICE, ISLICE)
    tab16 = jnp.pad(embeds_weight, ((0, 0), (0, 16 - embeds_weight.shape[1])))
    sc = _make_sc_kernel(batch, bpw)
    out = sc(c2w, ids2, tab16)
    return out.reshape(batch, 4, 4)
